# Initial kernel scaffold; baseline (speedup 1.0000x reference)
import jax
import jax.numpy as jnp
from jax.experimental import pallas as pl


def kernel(hidden_states, Wg, W1, W3, W2):
    raise NotImplementedError("write your pallas kernel here")


# speedup vs baseline: 1.1075x; 1.1075x over previous
import functools

import jax
import jax.numpy as jnp
from jax.experimental import pallas as pl
from jax.experimental.pallas import tpu as pltpu


def _moe_dense_body(x_ref, wg_ref, w1_ref, w3_ref, w2_ref, out_ref, *, num_experts):
    e = pl.program_id(1)
    f = pl.program_id(2)
    x = x_ref[...]
    wg = wg_ref[...]
    logits = jax.lax.dot_general(x, wg, (((1,), (1,)), ((), ())),
                                 preferred_element_type=jnp.float32)
    m = jnp.max(logits, axis=1, keepdims=True)
    ex = jnp.exp(logits - m)
    p = ex / jnp.sum(ex, axis=1, keepdims=True)
    iota = jax.lax.broadcasted_iota(jnp.int32, p.shape, 1)
    v1 = jnp.max(p, axis=1, keepdims=True)
    e1 = jnp.min(jnp.where(p >= v1, iota, num_experts), axis=1, keepdims=True)
    pm = jnp.where(iota == e1, -1.0, p)
    v2 = jnp.max(pm, axis=1, keepdims=True)
    e2 = jnp.min(jnp.where(pm >= v2, iota, num_experts), axis=1, keepdims=True)
    w_sel = jnp.where(e1 == e, v1, 0.0) + jnp.where(e2 == e, v2, 0.0)
    we = w_sel / (v1 + v2)

    h1 = jax.lax.dot_general(x, w1_ref[0], (((1,), (1,)), ((), ())),
                             preferred_element_type=jnp.float32)
    a = h1 * (1.0 / (1.0 + jnp.exp(-h1)))
    b = jax.lax.dot_general(x, w3_ref[0], (((1,), (1,)), ((), ())),
                            preferred_element_type=jnp.float32)
    y = jax.lax.dot_general(a * b, w2_ref[0], (((1,), (1,)), ((), ())),
                            preferred_element_type=jnp.float32)
    contrib = y * we

    @pl.when((e == 0) & (f == 0))
    def _init():
        out_ref[...] = contrib

    @pl.when((e > 0) | (f > 0))
    def _acc():
        out_ref[...] += contrib


def _moe_dense(x, wg, w1, w3, w2, *, rows_per_block=512, ff_block=1024,
               interpret=False):
    t, d = x.shape
    num_experts, ff, _ = w1.shape
    rows = min(rows_per_block, t)
    ffb = min(ff_block, ff)
    grid = (t // rows, num_experts, ff // ffb)
    body = functools.partial(_moe_dense_body, num_experts=num_experts)
    return pl.pallas_call(
        body,
        grid=grid,
        in_specs=[
            pl.BlockSpec((rows, d), lambda r, e, f: (r, 0)),
            pl.BlockSpec((num_experts, d), lambda r, e, f: (0, 0)),
            pl.BlockSpec((1, ffb, d), lambda r, e, f: (e, f, 0)),
            pl.BlockSpec((1, ffb, d), lambda r, e, f: (e, f, 0)),
            pl.BlockSpec((1, d, ffb), lambda r, e, f: (e, 0, f)),
        ],
        out_specs=pl.BlockSpec((rows, d), lambda r, e, f: (r, 0)),
        out_shape=jax.ShapeDtypeStruct((t, d), jnp.float32),
        compiler_params=pltpu.CompilerParams(
            dimension_semantics=("parallel", "arbitrary", "arbitrary"),
        ),
        interpret=interpret,
    )(x, wg, w1, w3, w2)


def kernel(hidden_states, Wg, W1, W3, W2):
    bsz, seq, d = hidden_states.shape
    x = hidden_states.reshape(-1, d)
    out = _moe_dense(x, Wg, W1, W3, W2)
    return out.reshape(bsz, seq, d)


# speedup vs baseline: 1.3120x; 1.1846x over previous
import functools

import jax
import jax.numpy as jnp
from jax import lax
from jax.experimental import pallas as pl
from jax.experimental.pallas import tpu as pltpu
from jax.experimental.pallas import tpu_sc as plsc

TILE = 128
_NC, _NS = 2, 16
_NW = _NC * _NS


def _router_body(x_ref, wg_ref, w1o_ref, w2o_ref, e1o_ref, e2o_ref, *, ne):
    x = x_ref[...]
    logits = lax.dot_general(x, wg_ref[...], (((1,), (1,)), ((), ())),
                             preferred_element_type=jnp.float32)
    m = jnp.max(logits, axis=1, keepdims=True)
    ex = jnp.exp(logits - m)
    p = ex / jnp.sum(ex, axis=1, keepdims=True)
    iota = lax.broadcasted_iota(jnp.int32, p.shape, 1)
    v1 = jnp.max(p, axis=1, keepdims=True)
    e1 = jnp.min(jnp.where(p >= v1, iota, ne), axis=1, keepdims=True)
    pm = jnp.where(iota == e1, -1.0, p)
    v2 = jnp.max(pm, axis=1, keepdims=True)
    e2 = jnp.min(jnp.where(pm >= v2, iota, ne), axis=1, keepdims=True)
    s = v1 + v2
    w1o_ref[...] = v1 / s
    w2o_ref[...] = v2 / s
    e1o_ref[...] = e1
    e2o_ref[...] = e2


def _router(x, wg, *, interpret=False):
    t, d = x.shape
    ne = wg.shape[0]
    rows = min(1024, t)
    grid = (t // rows,)
    body = functools.partial(_router_body, ne=ne)
    return pl.pallas_call(
        body,
        grid=grid,
        in_specs=[
            pl.BlockSpec((rows, d), lambda r: (r, 0)),
            pl.BlockSpec((ne, d), lambda r: (0, 0)),
        ],
        out_specs=[
            pl.BlockSpec((rows, 1), lambda r: (r, 0)),
            pl.BlockSpec((rows, 1), lambda r: (r, 0)),
            pl.BlockSpec((rows, 1), lambda r: (r, 0)),
            pl.BlockSpec((rows, 1), lambda r: (r, 0)),
        ],
        out_shape=[
            jax.ShapeDtypeStruct((t, 1), jnp.float32),
            jax.ShapeDtypeStruct((t, 1), jnp.float32),
            jax.ShapeDtypeStruct((t, 1), jnp.int32),
            jax.ShapeDtypeStruct((t, 1), jnp.int32),
        ],
        interpret=interpret,
    )(x, wg)


def _scan_rows(c, t):
    k = 1
    while k < t:
        pad = jnp.zeros((k,) + c.shape[1:], c.dtype)
        c = c + jnp.concatenate([pad, c[:-k]], axis=0)
        k *= 2
    return c


def _scan_lanes(c, n):
    k = 1
    while k < n:
        pad = jnp.zeros(c.shape[:1] + (k,), c.dtype)
        c = c + jnp.concatenate([pad, c[:, :-k]], axis=1)
        k *= 2
    return c


def _plan_body(e1_ref, e2_ref, dest_ref, te_ref, *, t, ne, nt):
    e1 = e1_ref[...]
    e2 = e2_ref[...]
    io_e1 = lax.broadcasted_iota(jnp.int32, (t, ne), 1)
    oh1 = (e1 == io_e1).astype(jnp.int32)
    oh2 = (e2 == io_e1).astype(jnp.int32)
    c1 = _scan_rows(oh1, t)
    c2 = _scan_rows(oh2, t)
    ex1 = c1 - oh1
    ex2 = c2 - oh2
    cnt1 = c1[t - 1:t, :]
    cnt = cnt1 + c2[t - 1:t, :]
    pc = ((cnt + (TILE - 1)) // TILE) * TILE
    off = _scan_lanes(pc, ne) - pc
    rank1 = jnp.sum(oh1 * ex1, axis=1, keepdims=True)
    rank2 = jnp.sum(oh2 * (cnt1 + ex2), axis=1, keepdims=True)
    base1 = jnp.sum(oh1 * off, axis=1, keepdims=True)
    base2 = jnp.sum(oh2 * off, axis=1, keepdims=True)
    dest_ref[...] = jnp.concatenate([base1 + rank1, base2 + rank2], axis=0)
    endc = off + pc
    tid = lax.broadcasted_iota(jnp.int32, (nt, ne), 0)
    te = jnp.sum((tid * TILE >= endc).astype(jnp.int32), axis=1, keepdims=True)
    te_ref[...] = jnp.minimum(te, ne - 1)


def _plan(e1, e2, ne, nt, *, interpret=False):
    t = e1.shape[0]
    body = functools.partial(_plan_body, t=t, ne=ne, nt=nt)
    return pl.pallas_call(
        body,
        out_shape=[
            jax.ShapeDtypeStruct((2 * t, 1), jnp.int32),
            jax.ShapeDtypeStruct((nt, 1), jnp.int32),
        ],
        interpret=interpret,
    )(e1, e2)


def _sc_scatter_rows(x, dest, np_rows):
    t, d = x.shape
    per_w = (2 * t) // _NW
    chunk = 32
    mesh = plsc.VectorSubcoreMesh(core_axis_name="c", subcore_axis_name="s")

    @functools.partial(
        pl.kernel, mesh=mesh,
        out_type=jax.ShapeDtypeStruct((np_rows, d), jnp.float32),
        scratch_types=[
            pltpu.VMEM((chunk,), jnp.int32),
            pltpu.VMEM((chunk, d), jnp.float32),
            pltpu.SemaphoreType.DMA,
        ],
    )
    def k(x_hbm, dest_hbm, xs_hbm, idx_v, rows_v, sem):
        wid = lax.axis_index("s") * _NC + lax.axis_index("c")
        for c in range(per_w // chunk):
            j0 = wid * per_w + c * chunk
            r0 = lax.rem(j0, t)
            pltpu.sync_copy(dest_hbm.at[pl.ds(j0, chunk)], idx_v)
            pltpu.sync_copy(x_hbm.at[pl.ds(r0, chunk)], rows_v)
            pltpu.async_copy(rows_v, xs_hbm.at[idx_v], sem).wait()

    return k(x, dest)


def _sc_gather_rows(ys, d1, d2):
    t = d1.shape[0]
    d = ys.shape[1]
    per_w = t // _NW
    chunk = 32
    mesh = plsc.VectorSubcoreMesh(core_axis_name="c", subcore_axis_name="s")

    @functools.partial(
        pl.kernel, mesh=mesh,
        out_type=[
            jax.ShapeDtypeStruct((t, d), jnp.float32),
            jax.ShapeDtypeStruct((t, d), jnp.float32),
        ],
        scratch_types=[
            pltpu.VMEM((chunk,), jnp.int32),
            pltpu.VMEM((chunk,), jnp.int32),
            pltpu.VMEM((chunk, d), jnp.float32),
            pltpu.VMEM((chunk, d), jnp.float32),
            pltpu.SemaphoreType.DMA,
            pltpu.SemaphoreType.DMA,
        ],
    )
    def k(ys_hbm, d1_hbm, d2_hbm, y1_hbm, y2_hbm,
          i1_v, i2_v, ra_v, rb_v, sem1, sem2):
        wid = lax.axis_index("s") * _NC + lax.axis_index("c")
        for c in range(per_w // chunk):
            base = wid * per_w + c * chunk
            pltpu.sync_copy(d1_hbm.at[pl.ds(base, chunk)], i1_v)
            cp1 = pltpu.async_copy(ys_hbm.at[i1_v], ra_v, sem1)
            pltpu.sync_copy(d2_hbm.at[pl.ds(base, chunk)], i2_v)
            cp2 = pltpu.async_copy(ys_hbm.at[i2_v], rb_v, sem2)
            cp1.wait()
            pltpu.sync_copy(ra_v, y1_hbm.at[pl.ds(base, chunk)])
            cp2.wait()
            pltpu.sync_copy(rb_v, y2_hbm.at[pl.ds(base, chunk)])

    return k(ys, d1, d2)


def _gmlp_body(te_ref, xs_ref, w1_ref, w3_ref, w2_ref, ys_ref):
    h = xs_ref[...]
    a = lax.dot_general(h, w1_ref[0], (((1,), (1,)), ((), ())),
                        preferred_element_type=jnp.float32)
    a = a * (1.0 / (1.0 + jnp.exp(-a)))
    b = lax.dot_general(h, w3_ref[0], (((1,), (1,)), ((), ())),
                        preferred_element_type=jnp.float32)
    y = lax.dot_general(a * b, w2_ref[0], (((1,), (1,)), ((), ())),
                        preferred_element_type=jnp.float32)
    ys_ref[...] = y


def _gmlp(te, xs, w1, w3, w2, *, interpret=False):
    np_rows, d = xs.shape
    ne, ff, _ = w1.shape
    nt = np_rows // TILE
    grid_spec = pltpu.PrefetchScalarGridSpec(
        num_scalar_prefetch=1,
        grid=(nt,),
        in_specs=[
            pl.BlockSpec((TILE, d), lambda i, te_r: (i, 0)),
            pl.BlockSpec((1, ff, d), lambda i, te_r: (te_r[i], 0, 0)),
            pl.BlockSpec((1, ff, d), lambda i, te_r: (te_r[i], 0, 0)),
            pl.BlockSpec((1, d, ff), lambda i, te_r: (te_r[i], 0, 0)),
        ],
        out_specs=pl.BlockSpec((TILE, d), lambda i, te_r: (i, 0)),
    )
    return pl.pallas_call(
        _gmlp_body,
        grid_spec=grid_spec,
        out_shape=jax.ShapeDtypeStruct((np_rows, d), jnp.float32),
        compiler_params=pltpu.CompilerParams(
            dimension_semantics=("arbitrary",),
        ),
        interpret=interpret,
    )(te, xs, w1, w3, w2)


def _combine_body(w1_ref, w2_ref, y1_ref, y2_ref, out_ref):
    out_ref[...] = y1_ref[...] * w1_ref[...] + y2_ref[...] * w2_ref[...]


def _combine(w1n, w2n, y1, y2, *, interpret=False):
    t, d = y1.shape
    rows = min(1024, t)
    return pl.pallas_call(
        _combine_body,
        grid=(t // rows,),
        in_specs=[
            pl.BlockSpec((rows, 1), lambda r: (r, 0)),
            pl.BlockSpec((rows, 1), lambda r: (r, 0)),
            pl.BlockSpec((rows, d), lambda r: (r, 0)),
            pl.BlockSpec((rows, d), lambda r: (r, 0)),
        ],
        out_specs=pl.BlockSpec((rows, d), lambda r: (r, 0)),
        out_shape=jax.ShapeDtypeStruct((t, d), jnp.float32),
        interpret=interpret,
    )(w1n, w2n, y1, y2)


def kernel(hidden_states, Wg, W1, W3, W2):
    bsz, seq, d = hidden_states.shape
    ne = Wg.shape[0]
    x = hidden_states.reshape(-1, d)
    t = x.shape[0]
    np_rows = 2 * t + ne * TILE
    nt = np_rows // TILE

    w1n, w2n, e1, e2 = _router(x, Wg)
    dest, te = _plan(e1, e2, ne, nt)
    dest_flat = dest.reshape(-1)
    xs = _sc_scatter_rows(x, dest_flat, np_rows)
    ys = _gmlp(te.reshape(-1), xs, W1, W3, W2)
    y1, y2 = _sc_gather_rows(ys, dest_flat[:t], dest_flat[t:])
    out = _combine(w1n, w2n, y1, y2)
    return out.reshape(bsz, seq, d)


# speedup vs baseline: 2.0160x; 1.5366x over previous
import functools

import jax
import jax.numpy as jnp
from jax import lax
from jax.experimental import pallas as pl
from jax.experimental.pallas import tpu as pltpu
from jax.experimental.pallas import tpu_sc as plsc

TILE = 256
_NC, _NS = 2, 16
_NW = _NC * _NS


def _router_body(x_ref, wg_ref, w1o_ref, w2o_ref, e1o_ref, e2o_ref, *, ne):
    x = x_ref[...]
    logits = lax.dot_general(x, wg_ref[...], (((1,), (1,)), ((), ())),
                             preferred_element_type=jnp.float32)
    m = jnp.max(logits, axis=1, keepdims=True)
    ex = jnp.exp(logits - m)
    p = ex / jnp.sum(ex, axis=1, keepdims=True)
    iota = lax.broadcasted_iota(jnp.int32, p.shape, 1)
    v1 = jnp.max(p, axis=1, keepdims=True)
    e1 = jnp.min(jnp.where(p >= v1, iota, ne), axis=1, keepdims=True)
    pm = jnp.where(iota == e1, -1.0, p)
    v2 = jnp.max(pm, axis=1, keepdims=True)
    e2 = jnp.min(jnp.where(pm >= v2, iota, ne), axis=1, keepdims=True)
    s = v1 + v2
    w1o_ref[...] = v1 / s
    w2o_ref[...] = v2 / s
    e1o_ref[...] = e1
    e2o_ref[...] = e2


def _router(x, wg, *, interpret=False):
    t, d = x.shape
    ne = wg.shape[0]
    rows = min(1024, t)
    grid = (t // rows,)
    body = functools.partial(_router_body, ne=ne)
    return pl.pallas_call(
        body,
        grid=grid,
        in_specs=[
            pl.BlockSpec((rows, d), lambda r: (r, 0)),
            pl.BlockSpec((ne, d), lambda r: (0, 0)),
        ],
        out_specs=[
            pl.BlockSpec((rows, 1), lambda r: (r, 0)),
            pl.BlockSpec((rows, 1), lambda r: (r, 0)),
            pl.BlockSpec((rows, 1), lambda r: (r, 0)),
            pl.BlockSpec((rows, 1), lambda r: (r, 0)),
        ],
        out_shape=[
            jax.ShapeDtypeStruct((t, 1), jnp.float32),
            jax.ShapeDtypeStruct((t, 1), jnp.float32),
            jax.ShapeDtypeStruct((t, 1), jnp.int32),
            jax.ShapeDtypeStruct((t, 1), jnp.int32),
        ],
        interpret=interpret,
    )(x, wg)


def _scan_rows(c, t):
    k = 1
    while k < t:
        pad = jnp.zeros((k,) + c.shape[1:], c.dtype)
        c = c + jnp.concatenate([pad, c[:-k]], axis=0)
        k *= 2
    return c


def _scan_lanes(c, n):
    k = 1
    while k < n:
        pad = jnp.zeros(c.shape[:1] + (k,), c.dtype)
        c = c + jnp.concatenate([pad, c[:, :-k]], axis=1)
        k *= 2
    return c


def _plan_body(e1_ref, e2_ref, dest_ref, te_ref, *, t, ne, nt):
    e1 = e1_ref[...]
    e2 = e2_ref[...]
    io_e1 = lax.broadcasted_iota(jnp.int32, (t, ne), 1)
    oh1 = (e1 == io_e1).astype(jnp.int32)
    oh2 = (e2 == io_e1).astype(jnp.int32)
    c1 = _scan_rows(oh1, t)
    c2 = _scan_rows(oh2, t)
    ex1 = c1 - oh1
    ex2 = c2 - oh2
    cnt1 = c1[t - 1:t, :]
    cnt = cnt1 + c2[t - 1:t, :]
    pc = ((cnt + (TILE - 1)) // TILE) * TILE
    off = _scan_lanes(pc, ne) - pc
    rank1 = jnp.sum(oh1 * ex1, axis=1, keepdims=True)
    rank2 = jnp.sum(oh2 * (cnt1 + ex2), axis=1, keepdims=True)
    base1 = jnp.sum(oh1 * off, axis=1, keepdims=True)
    base2 = jnp.sum(oh2 * off, axis=1, keepdims=True)
    dest_ref[...] = jnp.concatenate([base1 + rank1, base2 + rank2], axis=0)
    endc = off + pc
    tid = lax.broadcasted_iota(jnp.int32, (nt, ne), 0)
    te = jnp.sum((tid * TILE >= endc).astype(jnp.int32), axis=1, keepdims=True)
    te_ref[...] = jnp.minimum(te, ne - 1)


def _plan(e1, e2, ne, nt, *, interpret=False):
    t = e1.shape[0]
    body = functools.partial(_plan_body, t=t, ne=ne, nt=nt)
    return pl.pallas_call(
        body,
        out_shape=[
            jax.ShapeDtypeStruct((2 * t, 1), jnp.int32),
            jax.ShapeDtypeStruct((nt, 1), jnp.int32),
        ],
        interpret=interpret,
    )(e1, e2)


def _sc_scatter_rows(x, dest, np_rows):
    t, d = x.shape
    per_w = (2 * t) // _NW
    chunk = 32
    mesh = plsc.VectorSubcoreMesh(core_axis_name="c", subcore_axis_name="s")

    @functools.partial(
        pl.kernel, mesh=mesh,
        out_type=jax.ShapeDtypeStruct((np_rows, d), jnp.float32),
        scratch_types=[
            pltpu.VMEM((chunk,), jnp.int32),
            pltpu.VMEM((chunk, d), jnp.float32),
            pltpu.SemaphoreType.DMA,
        ],
    )
    def k(x_hbm, dest_hbm, xs_hbm, idx_v, rows_v, sem):
        wid = lax.axis_index("s") * _NC + lax.axis_index("c")
        for c in range(per_w // chunk):
            j0 = wid * per_w + c * chunk
            r0 = lax.rem(j0, t)
            pltpu.sync_copy(dest_hbm.at[pl.ds(j0, chunk)], idx_v)
            pltpu.sync_copy(x_hbm.at[pl.ds(r0, chunk)], rows_v)
            pltpu.async_copy(rows_v, xs_hbm.at[idx_v], sem).wait()

    return k(x, dest)


def _sc_gather_rows(ys, d1, d2):
    t = d1.shape[0]
    d = ys.shape[1]
    per_w = t // _NW
    chunk = 32
    mesh = plsc.VectorSubcoreMesh(core_axis_name="c", subcore_axis_name="s")

    @functools.partial(
        pl.kernel, mesh=mesh,
        out_type=[
            jax.ShapeDtypeStruct((t, d), jnp.float32),
            jax.ShapeDtypeStruct((t, d), jnp.float32),
        ],
        scratch_types=[
            pltpu.VMEM((chunk,), jnp.int32),
            pltpu.VMEM((chunk,), jnp.int32),
            pltpu.VMEM((chunk, d), jnp.float32),
            pltpu.VMEM((chunk, d), jnp.float32),
            pltpu.SemaphoreType.DMA,
            pltpu.SemaphoreType.DMA,
        ],
    )
    def k(ys_hbm, d1_hbm, d2_hbm, y1_hbm, y2_hbm,
          i1_v, i2_v, ra_v, rb_v, sem1, sem2):
        wid = lax.axis_index("s") * _NC + lax.axis_index("c")
        for c in range(per_w // chunk):
            base = wid * per_w + c * chunk
            pltpu.sync_copy(d1_hbm.at[pl.ds(base, chunk)], i1_v)
            cp1 = pltpu.async_copy(ys_hbm.at[i1_v], ra_v, sem1)
            pltpu.sync_copy(d2_hbm.at[pl.ds(base, chunk)], i2_v)
            cp2 = pltpu.async_copy(ys_hbm.at[i2_v], rb_v, sem2)
            cp1.wait()
            pltpu.sync_copy(ra_v, y1_hbm.at[pl.ds(base, chunk)])
            cp2.wait()
            pltpu.sync_copy(rb_v, y2_hbm.at[pl.ds(base, chunk)])

    return k(ys, d1, d2)


def _gmlp_body(te_ref, xs_ref, w1_ref, w3_ref, w2_ref, ys_ref):
    h = xs_ref[...]
    a = lax.dot_general(h, w1_ref[0], (((1,), (1,)), ((), ())),
                        preferred_element_type=jnp.float32)
    a = a * (1.0 / (1.0 + jnp.exp(-a)))
    b = lax.dot_general(h, w3_ref[0], (((1,), (1,)), ((), ())),
                        preferred_element_type=jnp.float32)
    y = lax.dot_general(a * b, w2_ref[0], (((1,), (1,)), ((), ())),
                        preferred_element_type=jnp.float32)
    ys_ref[...] = y


def _gmlp(te, xs, w1, w3, w2, *, interpret=False):
    np_rows, d = xs.shape
    ne, ff, _ = w1.shape
    nt = np_rows // TILE
    grid_spec = pltpu.PrefetchScalarGridSpec(
        num_scalar_prefetch=1,
        grid=(nt,),
        in_specs=[
            pl.BlockSpec((TILE, d), lambda i, te_r: (i, 0)),
            pl.BlockSpec((1, ff, d), lambda i, te_r: (te_r[i], 0, 0)),
            pl.BlockSpec((1, ff, d), lambda i, te_r: (te_r[i], 0, 0)),
            pl.BlockSpec((1, d, ff), lambda i, te_r: (te_r[i], 0, 0)),
        ],
        out_specs=pl.BlockSpec((TILE, d), lambda i, te_r: (i, 0)),
    )
    return pl.pallas_call(
        _gmlp_body,
        grid_spec=grid_spec,
        out_shape=jax.ShapeDtypeStruct((np_rows, d), jnp.float32),
        compiler_params=pltpu.CompilerParams(
            dimension_semantics=("arbitrary",),
        ),
        interpret=interpret,
    )(te, xs, w1, w3, w2)


def _combine_body(w1_ref, w2_ref, y1_ref, y2_ref, out_ref):
    out_ref[...] = y1_ref[...] * w1_ref[...] + y2_ref[...] * w2_ref[...]


def _combine(w1n, w2n, y1, y2, *, interpret=False):
    t, d = y1.shape
    rows = min(1024, t)
    return pl.pallas_call(
        _combine_body,
        grid=(t // rows,),
        in_specs=[
            pl.BlockSpec((rows, 1), lambda r: (r, 0)),
            pl.BlockSpec((rows, 1), lambda r: (r, 0)),
            pl.BlockSpec((rows, d), lambda r: (r, 0)),
            pl.BlockSpec((rows, d), lambda r: (r, 0)),
        ],
        out_specs=pl.BlockSpec((rows, d), lambda r: (r, 0)),
        out_shape=jax.ShapeDtypeStruct((t, d), jnp.float32),
        interpret=interpret,
    )(w1n, w2n, y1, y2)


def kernel(hidden_states, Wg, W1, W3, W2):
    bsz, seq, d = hidden_states.shape
    ne = Wg.shape[0]
    x = hidden_states.reshape(-1, d)
    t = x.shape[0]
    np_rows = 2 * t + ne * TILE
    nt = np_rows // TILE

    w1n, w2n, e1, e2 = _router(x, Wg)
    dest, te = _plan(e1, e2, ne, nt)
    dest_flat = dest.reshape(-1)
    xs = _sc_scatter_rows(x, dest_flat, np_rows)
    ys = _gmlp(te.reshape(-1), xs, W1, W3, W2)
    y1, y2 = _sc_gather_rows(ys, dest_flat[:t], dest_flat[t:])
    out = _combine(w1n, w2n, y1, y2)
    return out.reshape(bsz, seq, d)
